# baseline (speedup 1.0000x reference)
import functools

import numpy as np

import jax
import jax.numpy as jnp
from jax import lax
from jax.experimental import pallas as pl
from jax.experimental.pallas import tpu as pltpu
from jax.experimental.pallas import tpu_sc as plsc

N_NODES = 10000
N_EDGES = 320000

NC, NS = 2, 16
NW = NC * NS
CHUNK = 128
NROWS_E = N_EDGES // CHUNK
BASE_CHUNKS = NROWS_E // NW
EXTRA_FROM = NW - (NROWS_E - BASE_CHUNKS * NW)
N_ACC = 10000
ZROWS = 632
WB_ROWS = 624


def _tile_chunks(wid):
  cs = BASE_CHUNKS * wid + jnp.maximum(wid - EXTRA_FROM, 0)
  nct = BASE_CHUNKS + (wid >= EXTRA_FROM).astype(jnp.int32)
  return cs, nct


def _zero_acc(zrows, acc, s):
  @pl.when(s < NS - 1)
  def _full():
    pltpu.sync_copy(zrows, acc.at[pl.ds(s * ZROWS, ZROWS)])

  @pl.when(s == NS - 1)
  def _last():
    rem = N_ACC - (NS - 1) * ZROWS
    pltpu.sync_copy(zrows.at[pl.ds(0, rem)],
                    acc.at[pl.ds((NS - 1) * ZROWS, rem)])


def _write_back(acc, out, c, s):
  pltpu.sync_copy(acc.at[pl.ds(s * WB_ROWS, WB_ROWS)],
                  out.at[c, pl.ds(s * WB_ROWS, WB_ROWS)])

  @pl.when(s == NS - 1)
  def _tail():
    base = NS * WB_ROWS
    pltpu.sync_copy(acc.at[pl.ds(base, N_NODES - base)],
                    out.at[c, pl.ds(base, N_NODES - base)])


def _make_sc_cnt():
  mesh = plsc.VectorSubcoreMesh(core_axis_name="c", subcore_axis_name="s")

  @functools.partial(
      pl.kernel,
      out_type=jax.ShapeDtypeStruct((NC, N_NODES, 16), jnp.float32),
      mesh=mesh,
      scratch_types=[
          pltpu.VMEM((BASE_CHUNKS + 1, CHUNK), jnp.int32),
          pltpu.VMEM((CHUNK, 16), jnp.float32),
          pltpu.VMEM_SHARED((N_ACC, 16), jnp.float32),
          pltpu.SemaphoreType.DMA,
      ],
      compiler_params=pltpu.CompilerParams(use_tc_tiling_on_sc=False),
  )
  def cnt(edges, zrows, out, dst_v, ones_v, acc, sem):
    c = lax.axis_index("c")
    s = lax.axis_index("s")
    wid = c * NS + s
    cs, nct = _tile_chunks(wid)

    _zero_acc(zrows, acc, s)
    pltpu.sync_copy(edges.at[1, pl.ds(cs, BASE_CHUNKS + 1)], dst_v)

    def fill_ones(r, carry):
      ones_v[r] = jnp.ones((16,), jnp.float32)
      return carry

    lax.fori_loop(0, CHUNK, fill_ones, 0)
    plsc.subcore_barrier()

    K = 8

    def outer(io, carry):
      base = io * K
      for k in range(K):
        pltpu.async_copy(ones_v, acc.at[dst_v.at[base + k]], sem, add=True)
      for k in range(K):
        pltpu.make_async_copy(ones_v, acc.at[dst_v.at[base + k]], sem).wait()
      return carry

    nfull = BASE_CHUNKS // K
    lax.fori_loop(0, nfull, outer, 0)
    for k in range(nfull * K, BASE_CHUNKS + 1):
      @pl.when(k < nct)
      def _fire():
        pltpu.async_copy(ones_v, acc.at[dst_v.at[k]], sem, add=True)
    for k in range(nfull * K, BASE_CHUNKS + 1):
      @pl.when(k < nct)
      def _drain():
        pltpu.make_async_copy(ones_v, acc.at[dst_v.at[k]], sem).wait()

    plsc.subcore_barrier()
    _write_back(acc, out, c, s)

  return cnt


def _make_sc_agg(D, nbuf, staged):
  mesh = plsc.VectorSubcoreMesh(core_axis_name="c", subcore_axis_name="s")
  sbuf = 40 if staged else BASE_CHUNKS + 1

  @functools.partial(
      pl.kernel,
      out_type=jax.ShapeDtypeStruct((NC, N_NODES, D), jnp.float32),
      mesh=mesh,
      scratch_types=[
          pltpu.VMEM((sbuf, CHUNK), jnp.int32),
          pltpu.VMEM((sbuf, CHUNK), jnp.int32),
          [pltpu.VMEM((CHUNK, D), jnp.float32) for _ in range(nbuf)],
          pltpu.VMEM_SHARED((N_ACC, D), jnp.float32),
          [pltpu.SemaphoreType.DMA] * nbuf,
      ],
      compiler_params=pltpu.CompilerParams(use_tc_tiling_on_sc=False),
  )
  def agg(table, edges, zrows, out, src_v, dst_v, rows_v, acc, sems):
    c = lax.axis_index("c")
    s = lax.axis_index("s")
    wid = c * NS + s
    cs, nct = _tile_chunks(wid)

    _zero_acc(zrows, acc, s)

    def load_idx(row0):
      pltpu.sync_copy(edges.at[0, pl.ds(row0, sbuf)], src_v)
      pltpu.sync_copy(edges.at[1, pl.ds(row0, sbuf)], dst_v)

    load_idx(cs)
    plsc.subcore_barrier()

    def run_chunks(r0, count, count_max):
      for b in range(nbuf):
        pltpu.async_copy(table.at[src_v.at[r0 + b]], rows_v[b], sems[b])

      def outer(io, carry):
        for b in range(nbuf):
          k = io * nbuf + b
          pltpu.make_async_copy(table.at[src_v.at[r0 + k]], rows_v[b],
                                sems[b]).wait()
          pltpu.sync_copy(rows_v[b], acc.at[dst_v.at[r0 + k]], add=True)

          @pl.when(k + nbuf < count)
          def _refill():
            pltpu.async_copy(table.at[src_v.at[r0 + k + nbuf]], rows_v[b],
                             sems[b])
        return carry

      nfull = (count_max // nbuf) - 1
      lax.fori_loop(0, nfull, outer, 0)
      for k in range(nfull * nbuf, count_max):
        b = k % nbuf

        @pl.when(k < count)
        def _epi():
          pltpu.make_async_copy(table.at[src_v.at[r0 + k]], rows_v[b],
                                sems[b]).wait()
          pltpu.sync_copy(rows_v[b], acc.at[dst_v.at[r0 + k]], add=True)

          @pl.when(k + nbuf < count)
          def _refill():
            pltpu.async_copy(table.at[src_v.at[r0 + k + nbuf]], rows_v[b],
                             sems[b])

    if staged:
      run_chunks(jnp.int32(0), jnp.int32(40), 40)
      load_idx(cs + nct - 40)
      run_chunks(40 - (nct - 40), nct - 40, BASE_CHUNKS + 1 - 40)
    else:
      run_chunks(jnp.int32(0), nct, BASE_CHUNKS + 1)

    plsc.subcore_barrier()
    _write_back(acc, out, c, s)

  return agg


_sc_cnt = _make_sc_cnt()
_sc_agg_128 = _make_sc_agg(128, nbuf=2, staged=True)
_sc_agg_16 = _make_sc_agg(16, nbuf=8, staged=False)


_TC_BLK = 10000
_GRID = N_NODES // _TC_BLK


def _tc1_body(s1_ref, ic_ref, x_ref, w1r_ref, b1_ref, w1t_ref, w2r_ref,
              w2t_ref, b2_ref, p2_ref, r2_ref):
  ic = ic_ref[...]
  agg = (s1_ref[0] + s1_ref[1]) * ic[:, :1]
  h1 = jnp.maximum(
      jnp.dot(agg, w1r_ref[...], preferred_element_type=jnp.float32)
      + b1_ref[...]
      + jnp.dot(x_ref[...], w1t_ref[...], preferred_element_type=jnp.float32),
      0.0)
  p2_ref[...] = jnp.dot(h1, w2r_ref[...], preferred_element_type=jnp.float32)
  r2_ref[...] = (
      jnp.dot(h1, w2t_ref[...], preferred_element_type=jnp.float32)
      + b2_ref[...])


def _tc2_body(s2_ref, r2_ref, ic_ref, wmr_ref, wlr_ref, wmt_ref, wlt_ref,
              bm_ref, bl_ref, p3_ref, r3_ref):
  h2 = jnp.maximum(
      (s2_ref[0] + s2_ref[1]) * ic_ref[:, :1] + r2_ref[...], 0.0)
  w3r = jnp.concatenate(
      [wmr_ref[...], wlr_ref[...],
       jnp.zeros((128, 12), jnp.float32)], axis=1)
  w3t = jnp.concatenate(
      [wmt_ref[...], wlt_ref[...],
       jnp.zeros((128, 12), jnp.float32)], axis=1)
  b3 = jnp.concatenate(
      [bm_ref[...], bl_ref[...], jnp.zeros((1, 12), jnp.float32)], axis=1)
  p3_ref[...] = jnp.dot(h2, w3r, preferred_element_type=jnp.float32)
  r3_ref[...] = jnp.dot(h2, w3t, preferred_element_type=jnp.float32) + b3


def _row_blk(shape_tail):
  return pl.BlockSpec((_TC_BLK,) + shape_tail,
                      lambda i: (i,) + (0,) * len(shape_tail))


def _part_blk(d):
  return pl.BlockSpec((NC, _TC_BLK, d), lambda i: (0, i, 0))


def _full_blk(shape):
  return pl.BlockSpec(shape, lambda i: (0,) * len(shape))


def kernel(x, W1_rel, b1, W1_root, W2_rel, b2, W2_root, Wmu_rel, bmu,
           Wmu_root, Wls_rel, bls, Wls_root, edge_index):
  e3 = edge_index.reshape(2, NROWS_E, CHUNK)

  z128 = jnp.zeros((ZROWS, 128), jnp.float32)
  z16 = jnp.zeros((ZROWS, 16), jnp.float32)

  cnt = _sc_cnt(e3, z16)
  ic = 1.0 / jnp.maximum(cnt[0, :, :8] + cnt[1, :, :8], 1.0)
  s1 = _sc_agg_128(x, e3, z128)

  p2, r2 = pl.pallas_call(
      _tc1_body,
      grid=(_GRID,),
      in_specs=[
          _part_blk(128),
          _row_blk((8,)),
          _row_blk((128,)),
          _full_blk((128, 256)),
          _full_blk((1, 256)),
          _full_blk((128, 256)),
          _full_blk((256, 128)),
          _full_blk((256, 128)),
          _full_blk((1, 128)),
      ],
      out_specs=[_row_blk((128,)), _row_blk((128,))],
      out_shape=[
          jax.ShapeDtypeStruct((N_NODES, 128), jnp.float32),
          jax.ShapeDtypeStruct((N_NODES, 128), jnp.float32),
      ],
  )(s1, ic, x, W1_rel, b1.reshape(1, 256), W1_root, W2_rel, W2_root,
    b2.reshape(1, 128))

  s2 = _sc_agg_128(p2, e3, z128)

  p3, r3 = pl.pallas_call(
      _tc2_body,
      grid=(_GRID,),
      in_specs=[
          _part_blk(128),
          _row_blk((128,)),
          _row_blk((8,)),
          _full_blk((128, 2)),
          _full_blk((128, 2)),
          _full_blk((128, 2)),
          _full_blk((128, 2)),
          _full_blk((1, 2)),
          _full_blk((1, 2)),
      ],
      out_specs=[_row_blk((16,)), _row_blk((16,))],
      out_shape=[
          jax.ShapeDtypeStruct((N_NODES, 16), jnp.float32),
          jax.ShapeDtypeStruct((N_NODES, 16), jnp.float32),
      ],
  )(s2, r2, ic, Wmu_rel, Wls_rel, Wmu_root, Wls_root,
    bmu.reshape(1, 2), bls.reshape(1, 2))

  s3 = _sc_agg_16(p3, e3, z16)

  mu = (s3[0, :, 0:2] + s3[1, :, 0:2]) * ic[:, :1] + r3[:, 0:2]
  ls = (s3[0, :, 2:4] + s3[1, :, 2:4]) * ic[:, :1] + r3[:, 2:4]

  return mu, ls


# speedup vs baseline: 1.0129x; 1.0129x over previous
import functools

import numpy as np

import jax
import jax.numpy as jnp
from jax import lax
from jax.experimental import pallas as pl
from jax.experimental.pallas import tpu as pltpu
from jax.experimental.pallas import tpu_sc as plsc

N_NODES = 10000
N_EDGES = 320000

NC, NS = 2, 16
NW = NC * NS
CHUNK = 128
NROWS_E = N_EDGES // CHUNK
BASE_CHUNKS = NROWS_E // NW
EXTRA_FROM = NW - (NROWS_E - BASE_CHUNKS * NW)
N_ACC = 10000
ZROWS = 632
WB_ROWS = 624


def _tile_chunks(wid):
  cs = BASE_CHUNKS * wid + jnp.maximum(wid - EXTRA_FROM, 0)
  nct = BASE_CHUNKS + (wid >= EXTRA_FROM).astype(jnp.int32)
  return cs, nct


def _zero_acc(zrows, acc, s):
  @pl.when(s < NS - 1)
  def _full():
    pltpu.sync_copy(zrows, acc.at[pl.ds(s * ZROWS, ZROWS)])

  @pl.when(s == NS - 1)
  def _last():
    rem = N_ACC - (NS - 1) * ZROWS
    pltpu.sync_copy(zrows.at[pl.ds(0, rem)],
                    acc.at[pl.ds((NS - 1) * ZROWS, rem)])


def _write_back(acc, out, c, s):
  pltpu.sync_copy(acc.at[pl.ds(s * WB_ROWS, WB_ROWS)],
                  out.at[c, pl.ds(s * WB_ROWS, WB_ROWS)])

  @pl.when(s == NS - 1)
  def _tail():
    base = NS * WB_ROWS
    pltpu.sync_copy(acc.at[pl.ds(base, N_NODES - base)],
                    out.at[c, pl.ds(base, N_NODES - base)])


def _make_sc_cnt():
  mesh = plsc.VectorSubcoreMesh(core_axis_name="c", subcore_axis_name="s")

  @functools.partial(
      pl.kernel,
      out_type=jax.ShapeDtypeStruct((NC, N_NODES, 16), jnp.float32),
      mesh=mesh,
      scratch_types=[
          pltpu.VMEM((BASE_CHUNKS + 1, CHUNK), jnp.int32),
          pltpu.VMEM((CHUNK, 16), jnp.float32),
          pltpu.VMEM_SHARED((N_ACC, 16), jnp.float32),
          pltpu.SemaphoreType.DMA,
      ],
      compiler_params=pltpu.CompilerParams(use_tc_tiling_on_sc=False),
  )
  def cnt(edges, zrows, out, dst_v, ones_v, acc, sem):
    c = lax.axis_index("c")
    s = lax.axis_index("s")
    wid = c * NS + s
    cs, nct = _tile_chunks(wid)

    _zero_acc(zrows, acc, s)
    pltpu.sync_copy(edges.at[1, pl.ds(cs, BASE_CHUNKS + 1)], dst_v)

    def fill_ones(r, carry):
      ones_v[r] = jnp.ones((16,), jnp.float32)
      return carry

    lax.fori_loop(0, CHUNK, fill_ones, 0)
    plsc.subcore_barrier()

    K = 8

    def outer(io, carry):
      base = io * K
      for k in range(K):
        pltpu.async_copy(ones_v, acc.at[dst_v.at[base + k]], sem, add=True)
      for k in range(K):
        pltpu.make_async_copy(ones_v, acc.at[dst_v.at[base + k]], sem).wait()
      return carry

    nfull = BASE_CHUNKS // K
    lax.fori_loop(0, nfull, outer, 0)
    for k in range(nfull * K, BASE_CHUNKS + 1):
      @pl.when(k < nct)
      def _fire():
        pltpu.async_copy(ones_v, acc.at[dst_v.at[k]], sem, add=True)
    for k in range(nfull * K, BASE_CHUNKS + 1):
      @pl.when(k < nct)
      def _drain():
        pltpu.make_async_copy(ones_v, acc.at[dst_v.at[k]], sem).wait()

    plsc.subcore_barrier()
    _write_back(acc, out, c, s)

  return cnt


def _make_sc_agg(D, nbuf, staged):
  mesh = plsc.VectorSubcoreMesh(core_axis_name="c", subcore_axis_name="s")
  sbuf = 40 if staged else BASE_CHUNKS + 1

  @functools.partial(
      pl.kernel,
      out_type=jax.ShapeDtypeStruct((NC, N_NODES, D), jnp.float32),
      mesh=mesh,
      scratch_types=[
          pltpu.VMEM((sbuf, CHUNK), jnp.int32),
          pltpu.VMEM((sbuf, CHUNK), jnp.int32),
          [pltpu.VMEM((CHUNK, D), jnp.float32) for _ in range(nbuf)],
          pltpu.VMEM_SHARED((N_ACC, D), jnp.float32),
          [pltpu.SemaphoreType.DMA] * nbuf,
      ],
      compiler_params=pltpu.CompilerParams(use_tc_tiling_on_sc=False),
  )
  def agg(table, edges, zrows, out, src_v, dst_v, rows_v, acc, sems):
    c = lax.axis_index("c")
    s = lax.axis_index("s")
    wid = c * NS + s
    cs, nct = _tile_chunks(wid)

    _zero_acc(zrows, acc, s)

    def load_idx(row0):
      pltpu.sync_copy(edges.at[0, pl.ds(row0, sbuf)], src_v)
      pltpu.sync_copy(edges.at[1, pl.ds(row0, sbuf)], dst_v)

    load_idx(cs)
    plsc.subcore_barrier()

    def run_chunks(r0, count, count_max):
      for b in range(nbuf):
        pltpu.async_copy(table.at[src_v.at[r0 + b]], rows_v[b], sems[b])

      def outer(io, carry):
        for b in range(nbuf):
          k = io * nbuf + b
          pltpu.make_async_copy(table.at[src_v.at[r0 + k]], rows_v[b],
                                sems[b]).wait()
          pltpu.sync_copy(rows_v[b], acc.at[dst_v.at[r0 + k]], add=True)

          @pl.when(k + nbuf < count)
          def _refill():
            pltpu.async_copy(table.at[src_v.at[r0 + k + nbuf]], rows_v[b],
                             sems[b])
        return carry

      nfull = (count_max // nbuf) - 1
      lax.fori_loop(0, nfull, outer, 0)
      for k in range(nfull * nbuf, count_max):
        b = k % nbuf

        @pl.when(k < count)
        def _epi():
          pltpu.make_async_copy(table.at[src_v.at[r0 + k]], rows_v[b],
                                sems[b]).wait()
          pltpu.sync_copy(rows_v[b], acc.at[dst_v.at[r0 + k]], add=True)

          @pl.when(k + nbuf < count)
          def _refill():
            pltpu.async_copy(table.at[src_v.at[r0 + k + nbuf]], rows_v[b],
                             sems[b])

    if staged:
      run_chunks(jnp.int32(0), jnp.int32(40), 40)
      load_idx(cs + nct - 40)
      run_chunks(40 - (nct - 40), nct - 40, BASE_CHUNKS + 1 - 40)
    else:
      run_chunks(jnp.int32(0), nct, BASE_CHUNKS + 1)

    plsc.subcore_barrier()
    _write_back(acc, out, c, s)

  return agg


_sc_cnt = _make_sc_cnt()
_sc_agg_128 = _make_sc_agg(128, nbuf=2, staged=True)
_sc_agg_16 = _make_sc_agg(16, nbuf=8, staged=False)


_TC_BLK = 5000
_GRID = N_NODES // _TC_BLK


def _tc1_body(s1_ref, ic_ref, x_ref, w1r_ref, b1_ref, w1t_ref, w2r_ref,
              w2t_ref, b2_ref, p2_ref, r2_ref):
  ic = ic_ref[...]
  agg = (s1_ref[0] + s1_ref[1]) * ic[:, :1]
  h1 = jnp.maximum(
      jnp.dot(agg, w1r_ref[...], preferred_element_type=jnp.float32)
      + b1_ref[...]
      + jnp.dot(x_ref[...], w1t_ref[...], preferred_element_type=jnp.float32),
      0.0)
  p2_ref[...] = jnp.dot(h1, w2r_ref[...], preferred_element_type=jnp.float32)
  r2_ref[...] = (
      jnp.dot(h1, w2t_ref[...], preferred_element_type=jnp.float32)
      + b2_ref[...])


def _tc2_body(s2_ref, r2_ref, ic_ref, wmr_ref, wlr_ref, wmt_ref, wlt_ref,
              bm_ref, bl_ref, p3_ref, r3_ref):
  h2 = jnp.maximum(
      (s2_ref[0] + s2_ref[1]) * ic_ref[:, :1] + r2_ref[...], 0.0)
  w3r = jnp.concatenate(
      [wmr_ref[...], wlr_ref[...],
       jnp.zeros((128, 12), jnp.float32)], axis=1)
  w3t = jnp.concatenate(
      [wmt_ref[...], wlt_ref[...],
       jnp.zeros((128, 12), jnp.float32)], axis=1)
  b3 = jnp.concatenate(
      [bm_ref[...], bl_ref[...], jnp.zeros((1, 12), jnp.float32)], axis=1)
  p3_ref[...] = jnp.dot(h2, w3r, preferred_element_type=jnp.float32)
  r3_ref[...] = jnp.dot(h2, w3t, preferred_element_type=jnp.float32) + b3


def _row_blk(shape_tail):
  return pl.BlockSpec((_TC_BLK,) + shape_tail,
                      lambda i: (i,) + (0,) * len(shape_tail))


def _part_blk(d):
  return pl.BlockSpec((NC, _TC_BLK, d), lambda i: (0, i, 0))


def _full_blk(shape):
  return pl.BlockSpec(shape, lambda i: (0,) * len(shape))


def kernel(x, W1_rel, b1, W1_root, W2_rel, b2, W2_root, Wmu_rel, bmu,
           Wmu_root, Wls_rel, bls, Wls_root, edge_index):
  e3 = edge_index.reshape(2, NROWS_E, CHUNK)

  z128 = jnp.zeros((ZROWS, 128), jnp.float32)
  z16 = jnp.zeros((ZROWS, 16), jnp.float32)

  cnt = _sc_cnt(e3, z16)
  ic = 1.0 / jnp.maximum(cnt[0, :, :8] + cnt[1, :, :8], 1.0)
  s1 = _sc_agg_128(x, e3, z128)

  p2, r2 = pl.pallas_call(
      _tc1_body,
      grid=(_GRID,),
      in_specs=[
          _part_blk(128),
          _row_blk((8,)),
          _row_blk((128,)),
          _full_blk((128, 256)),
          _full_blk((1, 256)),
          _full_blk((128, 256)),
          _full_blk((256, 128)),
          _full_blk((256, 128)),
          _full_blk((1, 128)),
      ],
      out_specs=[_row_blk((128,)), _row_blk((128,))],
      out_shape=[
          jax.ShapeDtypeStruct((N_NODES, 128), jnp.float32),
          jax.ShapeDtypeStruct((N_NODES, 128), jnp.float32),
      ],
  )(s1, ic, x, W1_rel, b1.reshape(1, 256), W1_root, W2_rel, W2_root,
    b2.reshape(1, 128))

  s2 = _sc_agg_128(p2, e3, z128)

  p3, r3 = pl.pallas_call(
      _tc2_body,
      grid=(_GRID,),
      in_specs=[
          _part_blk(128),
          _row_blk((128,)),
          _row_blk((8,)),
          _full_blk((128, 2)),
          _full_blk((128, 2)),
          _full_blk((128, 2)),
          _full_blk((128, 2)),
          _full_blk((1, 2)),
          _full_blk((1, 2)),
      ],
      out_specs=[_row_blk((16,)), _row_blk((16,))],
      out_shape=[
          jax.ShapeDtypeStruct((N_NODES, 16), jnp.float32),
          jax.ShapeDtypeStruct((N_NODES, 16), jnp.float32),
      ],
  )(s2, r2, ic, Wmu_rel, Wls_rel, Wmu_root, Wls_root,
    bmu.reshape(1, 2), bls.reshape(1, 2))

  s3 = _sc_agg_16(p3, e3, z16)

  mu = (s3[0, :, 0:2] + s3[1, :, 0:2]) * ic[:, :1] + r3[:, 0:2]
  ls = (s3[0, :, 2:4] + s3[1, :, 2:4]) * ic[:, :1] + r3[:, 2:4]

  return mu, ls
